# baseline (speedup 1.0000x reference)
import functools

import jax
import jax.numpy as jnp
from jax import lax
from jax.experimental import pallas as pl
from jax.experimental.pallas import tpu as pltpu
from jax.experimental.pallas import tpu_sc as plsc

NC = 2
NS = 16
NW = NC * NS
L = 16

BATCH = 16384
DIM = 32
BPW = BATCH // NW
CHUNK = 128
NCHUNK = BPW // CHUNK
NGRP = BPW // L


def _body(users_r, items_r, user_table, item_table, out_hbm,
          uidx_v, iidx_v, urows_v, irows_v, out_v, sem):
    wid = lax.axis_index("s") * NC + lax.axis_index("c")

    pltpu.sync_copy(users_r.at[wid], uidx_v)
    pltpu.sync_copy(items_r.at[wid], iidx_v)

    copies = []
    for j in range(NCHUNK):
        copies.append(pltpu.async_copy(
            user_table.at[uidx_v.at[j]],
            urows_v.at[pl.ds(j * CHUNK, CHUNK)], sem))
        copies.append(pltpu.async_copy(
            item_table.at[iidx_v.at[j]],
            irows_v.at[pl.ds(j * CHUNK, CHUNK)], sem))
    for c in copies:
        c.wait()

    iota = lax.iota(jnp.int32, L)

    def grp(g, carry):
        r_idx = g * L + iota
        acc = jnp.zeros((L,), jnp.float32)
        for d in range(DIM):
            c_idx = jnp.full((L,), d, jnp.int32)
            u = plsc.load_gather(urows_v, [r_idx, c_idx])
            v = plsc.load_gather(irows_v, [r_idx, c_idx])
            acc = acc + u * v
        rating = 1.0 / (1.0 + jnp.exp(-acc))
        out_v[pl.ds(g * L, L)] = rating
        return carry

    lax.fori_loop(0, NGRP, grp, 0)
    pltpu.sync_copy(out_v, out_hbm.at[wid])


@jax.jit
def _run(users_r, items_r, user_table, item_table):
    mesh = plsc.VectorSubcoreMesh(core_axis_name="c", subcore_axis_name="s")
    f = pl.kernel(
        _body,
        out_type=jax.ShapeDtypeStruct((NW, BPW), jnp.float32),
        mesh=mesh,
        scratch_types=[
            pltpu.VMEM((NCHUNK, CHUNK), jnp.int32),
            pltpu.VMEM((NCHUNK, CHUNK), jnp.int32),
            pltpu.VMEM((BPW, DIM), jnp.float32),
            pltpu.VMEM((BPW, DIM), jnp.float32),
            pltpu.VMEM((BPW,), jnp.float32),
            pltpu.SemaphoreType.DMA,
        ],
        compiler_params=pltpu.CompilerParams(
            needs_layout_passes=False, use_tc_tiling_on_sc=False),
    )
    return f(users_r, items_r, user_table, item_table)


def kernel(users, items, user_table, item_table):
    users_r = users.reshape(NW, NCHUNK, CHUNK)
    items_r = items.reshape(NW, NCHUNK, CHUNK)
    out = _run(users_r, items_r, user_table, item_table)
    return out.reshape(BATCH)
